# baseline (speedup 1.0000x reference)
import dataclasses
import functools

import jax
import jax.numpy as jnp
from jax import lax
from jax.experimental import pallas as pl
from jax.experimental.pallas import tpu as pltpu
from jax.experimental.pallas import tpu_sc as plsc

N = 10000
D = 128
E = 320000
CHUNK = 128
NCORE = 2
NSUB = 16
EDGES_PER_CORE = E // NCORE
CHUNKS_PER_CORE = EDGES_PER_CORE // CHUNK
CHUNKS_BASE = CHUNKS_PER_CORE // NSUB
CHUNKS_REM = CHUNKS_PER_CORE - CHUNKS_BASE * NSUB
NPAD = 10240
ROWS_PER_SUB = NPAD // NSUB
ZROWS = 128
CH_TOTAL = NCORE * CHUNKS_PER_CORE
CH_PAD = 2560
PHASE = 40
IDXROWS = PHASE + 8

def _mesh():
    return plsc.VectorSubcoreMesh(core_axis_name="c", subcore_axis_name="s")


def _my_chunks(sid):
    start = sid * CHUNKS_BASE + jnp.minimum(sid, CHUNKS_REM)
    count = CHUNKS_BASE + jnp.where(sid < CHUNKS_REM, 1, 0)
    return start, count


DEG_BLOCKS = NPAD // CHUNK
DEG_BLK_PER_SUB = DEG_BLOCKS // NSUB
DEG_IDXROWS = 80 + 8


@jax.jit
def _sc_degree(ei2d):
    cp = pltpu.CompilerParams()
    if "needs_layout_passes" in pltpu.CompilerParams.__dataclass_fields__:
        cp = dataclasses.replace(cp, needs_layout_passes=False)

    @functools.partial(
        pl.kernel,
        out_type=jax.ShapeDtypeStruct((NCORE, NPAD, CHUNK), jnp.float32),
        mesh=_mesh(),
        compiler_params=cp,
        scratch_types=[
            pltpu.VMEM((DEG_IDXROWS, CHUNK), jnp.int32),
            pltpu.VMEM((NPAD,), jnp.float32),
            pltpu.VMEM((NSUB, CHUNK), jnp.float32),
            pltpu.VMEM((CHUNK, CHUNK), jnp.float32),
            pltpu.VMEM_SHARED((NSUB, NPAD), jnp.float32),
            pltpu.SemaphoreType.DMA,
        ],
    )
    def k(ei_hbm, out_hbm, dv, hist, red, tbuf, stag, semi):
        cid = lax.axis_index("c")
        sid = lax.axis_index("s")
        start, count = _my_chunks(sid)
        cbase = cid * CHUNKS_PER_CORE + start
        abase = (cbase // 8) * 8
        ofs = cbase - abase

        pltpu.async_copy(ei_hbm.at[1, pl.ds(abase, DEG_IDXROWS)], dv, semi)

        @pl.loop(0, NPAD // 16)
        def _(i):
            hist[pl.ds(i * 16, 16)] = jnp.zeros((16,), jnp.float32)

        pltpu.make_async_copy(ei_hbm.at[1, pl.ds(abase, DEG_IDXROWS)], dv, semi).wait()

        ones = jnp.full((16,), 1.0, jnp.float32)

        @pl.loop(0, count)
        def _(j):
            for sl in range(CHUNK // 16):
                idxv = dv[ofs + j, pl.ds(sl * 16, 16)]
                plsc.addupdate_scatter(hist, [idxv], ones)

        pltpu.sync_copy(hist, stag.at[sid])
        plsc.subcore_barrier()

        cols0 = jnp.zeros((16,), jnp.int32)
        lane = jnp.arange(16, dtype=jnp.int32)

        @pl.loop(0, DEG_BLK_PER_SUB)
        def _(b):
            blk = sid * DEG_BLK_PER_SUB + b
            pltpu.sync_copy(stag.at[:, pl.ds(blk * CHUNK, CHUNK)], red)
            for sl in range(CHUNK // 16):
                v = red[0, pl.ds(sl * 16, 16)]
                for r in range(1, NSUB):
                    v = v + red[r, pl.ds(sl * 16, 16)]
                plsc.store_scatter(tbuf, [sl * 16 + lane, cols0], v)
            pltpu.sync_copy(tbuf, out_hbm.at[cid, pl.ds(blk * CHUNK, CHUNK)])

    return k(ei2d)


@jax.jit
def _sc_aggregate(h, ei2d):
    @functools.partial(
        pl.kernel,
        out_type=jax.ShapeDtypeStruct((NCORE, NPAD, D), jnp.float32),
        mesh=_mesh(),
        scratch_types=[
            pltpu.VMEM((IDXROWS, CHUNK), jnp.int32),
            pltpu.VMEM((IDXROWS, CHUNK), jnp.int32),
            pltpu.VMEM((CHUNK, D), jnp.float32),
            pltpu.VMEM((CHUNK, D), jnp.float32),
            pltpu.VMEM_SHARED((NPAD, D), jnp.float32),
            pltpu.SemaphoreType.DMA,
            pltpu.SemaphoreType.DMA,
            pltpu.SemaphoreType.DMA,
        ],
    )
    def k(h_hbm, ei_hbm, out_hbm, sv, dv, buf0, buf1, acc,
          semi, sem0, sem1):
        cid = lax.axis_index("c")
        sid = lax.axis_index("s")

        start, count = _my_chunks(sid)
        cbase = cid * CHUNKS_PER_CORE + start
        abase = (cbase // 8) * 8
        ofs = cbase - abase

        @pl.loop(0, ZROWS)
        def _(r):
            @pl.loop(0, D // 16)
            def _(cc):
                buf0[r, pl.ds(cc * 16, 16)] = jnp.zeros((16,), jnp.float32)

        @pl.loop(0, ROWS_PER_SUB // ZROWS)
        def _(j):
            pltpu.sync_copy(buf0, acc.at[pl.ds(sid * ROWS_PER_SUB + j * ZROWS, ZROWS)])

        plsc.subcore_barrier()

        def g_start(j, buf, sem):
            pltpu.async_copy(h_hbm.at[sv.at[ofs + j]], buf, sem)

        def g_wait(j, buf, sem):
            pltpu.make_async_copy(h_hbm.at[sv.at[ofs + j]], buf, sem).wait()

        def s_do(j, buf):
            pltpu.sync_copy(buf, acc.at[dv.at[ofs + j]], add=True)

        for p in range(2):
            pbase = p * PHASE
            pcount = jnp.minimum(count - pbase, PHASE)
            pltpu.async_copy(ei_hbm.at[0, pl.ds(abase + pbase, IDXROWS)], sv, semi)
            pltpu.async_copy(ei_hbm.at[1, pl.ds(abase + pbase, IDXROWS)], dv, semi)
            pltpu.make_async_copy(ei_hbm.at[0, pl.ds(abase + pbase, IDXROWS)], sv, semi).wait()
            pltpu.make_async_copy(ei_hbm.at[1, pl.ds(abase + pbase, IDXROWS)], dv, semi).wait()

            npairs = pcount // 2
            tail = pcount - 2 * npairs

            g_start(0, buf0, sem0)

            @pl.loop(0, npairs)
            def _(t):
                c0 = 2 * t
                c1 = c0 + 1
                g_start(c1, buf1, sem1)
                g_wait(c0, buf0, sem0)
                s_do(c0, buf0)

                @pl.when(c1 + 1 < pcount)
                def _():
                    g_start(c1 + 1, buf0, sem0)

                g_wait(c1, buf1, sem1)
                s_do(c1, buf1)

            @pl.when(tail == 1)
            def _():
                c = 2 * npairs
                g_wait(c, buf0, sem0)
                s_do(c, buf0)

        plsc.subcore_barrier()

        @pl.loop(0, ROWS_PER_SUB // ZROWS)
        def _(j):
            r0 = sid * ROWS_PER_SUB + j * ZROWS
            pltpu.sync_copy(acc.at[pl.ds(r0, ZROWS)], out_hbm.at[cid, pl.ds(r0, ZROWS)])

    return k(h, ei2d)


def _tc_prep(x, W0, degp):
    def body(x_ref, w_ref, degp_ref, dinv_ref, h_ref):
        deg = degp_ref[0, :N, 0:1] + degp_ref[1, :N, 0:1] + 1.0
        dinv = lax.rsqrt(deg)
        dinv_ref[...] = dinv
        h = jnp.dot(x_ref[...], w_ref[...], preferred_element_type=jnp.float32)
        h_ref[...] = h * dinv

    return pl.pallas_call(
        body,
        out_shape=(
            jax.ShapeDtypeStruct((N, 1), jnp.float32),
            jax.ShapeDtypeStruct((N, D), jnp.float32),
        ),
    )(x, W0, degp)


NB = 10
BR = N // NB


def _tc_combine(p, hp, dinv, b, g, be, resid, Wn, relu):
    have_res = resid is not None
    have_w = Wn is not None

    def body(*refs):
        it = iter(refs)
        p_ref = next(it)
        hp_ref = next(it)
        dinv_ref = next(it)
        b_ref = next(it)
        g_ref = next(it)
        be_ref = next(it)
        res_ref = next(it) if have_res else None
        w_ref = next(it) if have_w else None
        y_ref = next(it)
        hn_ref = next(it) if have_w else None
        s_scr = next(it)
        m_scr = next(it)

        ph = pl.program_id(0)
        i = pl.program_id(1)

        @pl.when(jnp.logical_and(ph == 0, i == 0))
        def _():
            m_scr[...] = jnp.zeros_like(m_scr)

        @pl.when(ph == 0)
        def _():
            s = (p_ref[0] + p_ref[1] + hp_ref[...]) * dinv_ref[...] + b_ref[...]
            s_scr[pl.ds(i * BR, BR), :] = s
            m_scr[0:1, :] += jnp.sum(s, axis=0, keepdims=True)
            m_scr[1:2, :] += jnp.sum(s * s, axis=0, keepdims=True)

        @pl.when(ph == 1)
        def _():
            mu = m_scr[0:1, :] * (1.0 / N)
            var = m_scr[1:2, :] * (1.0 / N) - mu * mu
            s = s_scr[pl.ds(i * BR, BR), :]
            y = g_ref[...] * (s - mu) * lax.rsqrt(var + 1e-5) + be_ref[...]
            if relu:
                y = jnp.maximum(y, 0.0)
            if have_res:
                y = y + res_ref[...]
            y_ref[...] = y
            if have_w:
                hn = jnp.dot(y, w_ref[...], preferred_element_type=jnp.float32)
                hn_ref[...] = hn * dinv_ref[...]

    def ix0(ph, i):
        return (0, jnp.where(ph == 0, i, 0), 0)

    def row0(ph, i):
        return (jnp.where(ph == 0, i, 0), 0)

    def row1(ph, i):
        return (jnp.where(ph == 1, i, 0), 0)

    def rowb(ph, i):
        return (i, 0)

    def const(ph, i):
        return (0, 0)

    in_specs = [
        pl.BlockSpec((2, BR, D), ix0),
        pl.BlockSpec((BR, D), row0),
        pl.BlockSpec((BR, 1), rowb),
        pl.BlockSpec((1, D), const),
        pl.BlockSpec((1, D), const),
        pl.BlockSpec((1, D), const),
    ]
    args = [p, hp, dinv, b.reshape(1, D), g.reshape(1, D), be.reshape(1, D)]
    if have_res:
        in_specs.append(pl.BlockSpec((BR, D), row1))
        args.append(resid)
    if have_w:
        in_specs.append(pl.BlockSpec((D, D), const))
        args.append(Wn)

    out_shape = [jax.ShapeDtypeStruct((N, D), jnp.float32)]
    out_specs = [pl.BlockSpec((BR, D), row1)]
    if have_w:
        out_shape.append(jax.ShapeDtypeStruct((N, D), jnp.float32))
        out_specs.append(pl.BlockSpec((BR, D), row1))

    res = pl.pallas_call(
        body,
        grid=(2, NB),
        in_specs=in_specs,
        out_specs=out_specs,
        out_shape=tuple(out_shape),
        scratch_shapes=[
            pltpu.VMEM((N, D), jnp.float32),
            pltpu.VMEM((2, D), jnp.float32),
        ],
    )(*args)
    return res if have_w else (res[0], None)


def kernel(x, edge_index, W0, b0, g0, be0, W1, b1, g1, be1, W2, b2, g2, be2):
    ei = edge_index.astype(jnp.int32)
    pad = CH_PAD * CHUNK - E
    ei2d = jnp.pad(ei, ((0, 0), (0, pad))).reshape(2, CH_PAD, CHUNK)

    degp = _sc_degree(ei2d)
    dinv, h0 = _tc_prep(x, W0, degp)

    p = _sc_aggregate(h0, ei2d)
    y0, h1 = _tc_combine(p, h0, dinv, b0, g0, be0, None, W1, relu=True)

    p = _sc_aggregate(h1, ei2d)
    y1, h2 = _tc_combine(p, h1, dinv, b1, g1, be1, y0, W2, relu=True)

    p = _sc_aggregate(h2, ei2d)
    y2, _ = _tc_combine(p, h2, dinv, b2, g2, be2, y1, None, relu=False)
    return y2


# speedup vs baseline: 1.0619x; 1.0619x over previous
import dataclasses
import functools

import jax
import jax.numpy as jnp
from jax import lax
from jax.experimental import pallas as pl
from jax.experimental.pallas import tpu as pltpu
from jax.experimental.pallas import tpu_sc as plsc

N = 10000
D = 128
E = 320000
CHUNK = 128
NCORE = 2
NSUB = 16
EDGES_PER_CORE = E // NCORE
CHUNKS_PER_CORE = EDGES_PER_CORE // CHUNK
CHUNKS_BASE = CHUNKS_PER_CORE // NSUB
CHUNKS_REM = CHUNKS_PER_CORE - CHUNKS_BASE * NSUB
NPAD = 10240
ROWS_PER_SUB = NPAD // NSUB
ZROWS = 128
CH_TOTAL = NCORE * CHUNKS_PER_CORE
CH_PAD = 2560
PHASE = 40
IDXROWS = PHASE + 8

def _mesh():
    return plsc.VectorSubcoreMesh(core_axis_name="c", subcore_axis_name="s")


def _my_chunks(sid):
    start = sid * CHUNKS_BASE + jnp.minimum(sid, CHUNKS_REM)
    count = CHUNKS_BASE + jnp.where(sid < CHUNKS_REM, 1, 0)
    return start, count


DEG_BLOCKS = NPAD // CHUNK
DEG_BLK_PER_SUB = DEG_BLOCKS // NSUB
DEG_IDXROWS = 80 + 8


@jax.jit
def _sc_degree(ei2d):
    cp = pltpu.CompilerParams()
    if "needs_layout_passes" in pltpu.CompilerParams.__dataclass_fields__:
        cp = dataclasses.replace(cp, needs_layout_passes=False)

    @functools.partial(
        pl.kernel,
        out_type=jax.ShapeDtypeStruct((NCORE, NPAD, CHUNK), jnp.float32),
        mesh=_mesh(),
        compiler_params=cp,
        scratch_types=[
            pltpu.VMEM((DEG_IDXROWS, CHUNK), jnp.int32),
            pltpu.VMEM((NPAD,), jnp.float32),
            pltpu.VMEM((NSUB, CHUNK), jnp.float32),
            pltpu.VMEM((CHUNK, CHUNK), jnp.float32),
            pltpu.VMEM_SHARED((NSUB, NPAD), jnp.float32),
            pltpu.SemaphoreType.DMA,
        ],
    )
    def k(ei_hbm, out_hbm, dv, hist, red, tbuf, stag, semi):
        cid = lax.axis_index("c")
        sid = lax.axis_index("s")
        start, count = _my_chunks(sid)
        cbase = cid * CHUNKS_PER_CORE + start
        abase = (cbase // 8) * 8
        ofs = cbase - abase

        pltpu.async_copy(ei_hbm.at[1, pl.ds(abase, DEG_IDXROWS)], dv, semi)

        @pl.loop(0, NPAD // 16)
        def _(i):
            hist[pl.ds(i * 16, 16)] = jnp.zeros((16,), jnp.float32)

        pltpu.make_async_copy(ei_hbm.at[1, pl.ds(abase, DEG_IDXROWS)], dv, semi).wait()

        ones = jnp.full((16,), 1.0, jnp.float32)

        @pl.loop(0, count)
        def _(j):
            for sl in range(CHUNK // 16):
                idxv = dv[ofs + j, pl.ds(sl * 16, 16)]
                plsc.addupdate_scatter(hist, [idxv], ones)

        pltpu.sync_copy(hist, stag.at[sid])
        plsc.subcore_barrier()

        cols0 = jnp.zeros((16,), jnp.int32)
        lane = jnp.arange(16, dtype=jnp.int32)

        @pl.loop(0, DEG_BLK_PER_SUB)
        def _(b):
            blk = sid * DEG_BLK_PER_SUB + b
            pltpu.sync_copy(stag.at[:, pl.ds(blk * CHUNK, CHUNK)], red)
            for sl in range(CHUNK // 16):
                v = red[0, pl.ds(sl * 16, 16)]
                for r in range(1, NSUB):
                    v = v + red[r, pl.ds(sl * 16, 16)]
                plsc.store_scatter(tbuf, [sl * 16 + lane, cols0], v)
            pltpu.sync_copy(tbuf, out_hbm.at[cid, pl.ds(blk * CHUNK, CHUNK)])

    return k(ei2d)


@jax.jit
def _sc_aggregate(h, ei2d):
    @functools.partial(
        pl.kernel,
        out_type=jax.ShapeDtypeStruct((NCORE, NPAD, D), jnp.float32),
        mesh=_mesh(),
        scratch_types=[
            pltpu.VMEM((IDXROWS, CHUNK), jnp.int32),
            pltpu.VMEM((IDXROWS, CHUNK), jnp.int32),
            pltpu.VMEM((CHUNK, D), jnp.float32),
            pltpu.VMEM((CHUNK, D), jnp.float32),
            pltpu.VMEM_SHARED((NPAD, D), jnp.float32),
            pltpu.SemaphoreType.DMA,
            pltpu.SemaphoreType.DMA,
            pltpu.SemaphoreType.DMA,
        ],
    )
    def k(h_hbm, ei_hbm, out_hbm, sv, dv, buf0, buf1, acc,
          semi, sem0, sem1):
        cid = lax.axis_index("c")
        sid = lax.axis_index("s")

        start, count = _my_chunks(sid)
        cbase = cid * CHUNKS_PER_CORE + start
        abase = (cbase // 8) * 8
        ofs = cbase - abase

        def g_start(j, buf, sem):
            pltpu.async_copy(h_hbm.at[sv.at[ofs + j]], buf, sem)

        def g_wait(j, buf, sem):
            pltpu.make_async_copy(h_hbm.at[sv.at[ofs + j]], buf, sem).wait()

        def s_do(j, buf):
            pltpu.sync_copy(buf, acc.at[dv.at[ofs + j]], add=True)

        pltpu.async_copy(ei_hbm.at[0, pl.ds(abase, IDXROWS)], sv, semi)
        pltpu.async_copy(ei_hbm.at[1, pl.ds(abase, IDXROWS)], dv, semi)

        @pl.loop(0, ZROWS)
        def _(r):
            @pl.loop(0, D // 16)
            def _(cc):
                buf1[r, pl.ds(cc * 16, 16)] = jnp.zeros((16,), jnp.float32)

        pltpu.make_async_copy(ei_hbm.at[0, pl.ds(abase, IDXROWS)], sv, semi).wait()
        pltpu.make_async_copy(ei_hbm.at[1, pl.ds(abase, IDXROWS)], dv, semi).wait()
        g_start(0, buf0, sem0)

        @pl.loop(0, ROWS_PER_SUB // ZROWS)
        def _(j):
            pltpu.sync_copy(buf1, acc.at[pl.ds(sid * ROWS_PER_SUB + j * ZROWS, ZROWS)])

        plsc.subcore_barrier()

        for p in range(2):
            pbase = p * PHASE
            pcount = jnp.minimum(count - pbase, PHASE)
            if p > 0:
                pltpu.async_copy(ei_hbm.at[0, pl.ds(abase + pbase, IDXROWS)], sv, semi)
                pltpu.async_copy(ei_hbm.at[1, pl.ds(abase + pbase, IDXROWS)], dv, semi)
                pltpu.make_async_copy(ei_hbm.at[0, pl.ds(abase + pbase, IDXROWS)], sv, semi).wait()
                pltpu.make_async_copy(ei_hbm.at[1, pl.ds(abase + pbase, IDXROWS)], dv, semi).wait()

            npairs = pcount // 2
            tail = pcount - 2 * npairs

            if p > 0:
                g_start(0, buf0, sem0)

            @pl.loop(0, npairs)
            def _(t):
                c0 = 2 * t
                c1 = c0 + 1
                g_start(c1, buf1, sem1)
                g_wait(c0, buf0, sem0)
                s_do(c0, buf0)

                @pl.when(c1 + 1 < pcount)
                def _():
                    g_start(c1 + 1, buf0, sem0)

                g_wait(c1, buf1, sem1)
                s_do(c1, buf1)

            @pl.when(tail == 1)
            def _():
                c = 2 * npairs
                g_wait(c, buf0, sem0)
                s_do(c, buf0)

        plsc.subcore_barrier()

        @pl.loop(0, ROWS_PER_SUB // ZROWS)
        def _(j):
            r0 = sid * ROWS_PER_SUB + j * ZROWS
            pltpu.sync_copy(acc.at[pl.ds(r0, ZROWS)], out_hbm.at[cid, pl.ds(r0, ZROWS)])

    return k(h, ei2d)


def _tc_prep(x, W0, degp):
    def body(x_ref, w_ref, degp_ref, dinv_ref, h_ref):
        deg = degp_ref[0, :N, 0:1] + degp_ref[1, :N, 0:1] + 1.0
        dinv = lax.rsqrt(deg)
        dinv_ref[...] = dinv
        h = jnp.dot(x_ref[...], w_ref[...], preferred_element_type=jnp.float32)
        h_ref[...] = h * dinv

    return pl.pallas_call(
        body,
        out_shape=(
            jax.ShapeDtypeStruct((N, 1), jnp.float32),
            jax.ShapeDtypeStruct((N, D), jnp.float32),
        ),
    )(x, W0, degp)


def _tc_combine(p, hp, dinv, b, g, be, resid, Wn, relu):
    have_res = resid is not None
    have_w = Wn is not None

    def body(*refs):
        it = iter(refs)
        p_ref = next(it)
        hp_ref = next(it)
        dinv_ref = next(it)
        b_ref = next(it)
        g_ref = next(it)
        be_ref = next(it)
        res_ref = next(it) if have_res else None
        w_ref = next(it) if have_w else None
        y_ref = next(it)
        hn_ref = next(it) if have_w else None

        dinv = dinv_ref[...]
        s = (p_ref[0, :N] + p_ref[1, :N] + hp_ref[...]) * dinv + b_ref[...]
        mu = jnp.mean(s, axis=0, keepdims=True)
        c = s - mu
        var = jnp.mean(c * c, axis=0, keepdims=True)
        y = g_ref[...] * c * lax.rsqrt(var + 1e-5) + be_ref[...]
        if relu:
            y = jnp.maximum(y, 0.0)
        if have_res:
            y = y + res_ref[...]
        y_ref[...] = y
        if have_w:
            hn = jnp.dot(y, w_ref[...], preferred_element_type=jnp.float32)
            hn_ref[...] = hn * dinv

    out_shape = [jax.ShapeDtypeStruct((N, D), jnp.float32)]
    if have_w:
        out_shape.append(jax.ShapeDtypeStruct((N, D), jnp.float32))
    args = [p, hp, dinv, b.reshape(1, D), g.reshape(1, D), be.reshape(1, D)]
    if have_res:
        args.append(resid)
    if have_w:
        args.append(Wn)
    res = pl.pallas_call(body, out_shape=tuple(out_shape))(*args)
    return res if have_w else (res[0], None)


def kernel(x, edge_index, W0, b0, g0, be0, W1, b1, g1, be1, W2, b2, g2, be2):
    ei = edge_index.astype(jnp.int32)
    pad = CH_PAD * CHUNK - E
    ei2d = jnp.pad(ei, ((0, 0), (0, pad))).reshape(2, CH_PAD, CHUNK)

    degp = _sc_degree(ei2d)
    dinv, h0 = _tc_prep(x, W0, degp)

    p = _sc_aggregate(h0, ei2d)
    y0, h1 = _tc_combine(p, h0, dinv, b0, g0, be0, None, W1, relu=True)

    p = _sc_aggregate(h1, ei2d)
    y1, h2 = _tc_combine(p, h1, dinv, b1, g1, be1, y0, W2, relu=True)

    p = _sc_aggregate(h2, ei2d)
    y2, _ = _tc_combine(p, h2, dinv, b2, g2, be2, y1, None, relu=False)
    return y2


# speedup vs baseline: 1.0632x; 1.0012x over previous
import dataclasses
import functools

import jax
import jax.numpy as jnp
from jax import lax
from jax.experimental import pallas as pl
from jax.experimental.pallas import tpu as pltpu
from jax.experimental.pallas import tpu_sc as plsc

N = 10000
D = 128
E = 320000
CHUNK = 128
NCORE = 2
NSUB = 16
EDGES_PER_CORE = E // NCORE
CHUNKS_PER_CORE = EDGES_PER_CORE // CHUNK
CHUNKS_BASE = CHUNKS_PER_CORE // NSUB
CHUNKS_REM = CHUNKS_PER_CORE - CHUNKS_BASE * NSUB
NPAD = 10240
ROWS_PER_SUB = NPAD // NSUB
ZROWS = 128
CH_TOTAL = NCORE * CHUNKS_PER_CORE
CH_PAD = 2560
PHASE = 40
IDXROWS = PHASE + 8

def _mesh():
    return plsc.VectorSubcoreMesh(core_axis_name="c", subcore_axis_name="s")


def _my_chunks(sid):
    start = sid * CHUNKS_BASE + jnp.minimum(sid, CHUNKS_REM)
    count = CHUNKS_BASE + jnp.where(sid < CHUNKS_REM, 1, 0)
    return start, count


DEG_BLOCKS = NPAD // CHUNK
DEG_BLK_PER_SUB = DEG_BLOCKS // NSUB
DEG_IDXROWS = 80 + 8


@jax.jit
def _sc_degree(ei2d):
    cp = pltpu.CompilerParams()
    if "needs_layout_passes" in pltpu.CompilerParams.__dataclass_fields__:
        cp = dataclasses.replace(cp, needs_layout_passes=False)

    @functools.partial(
        pl.kernel,
        out_type=jax.ShapeDtypeStruct((NCORE, NPAD, CHUNK), jnp.float32),
        mesh=_mesh(),
        compiler_params=cp,
        scratch_types=[
            pltpu.VMEM((DEG_IDXROWS, CHUNK), jnp.int32),
            pltpu.VMEM((NPAD,), jnp.float32),
            pltpu.VMEM((NSUB, CHUNK), jnp.float32),
            pltpu.VMEM((CHUNK, CHUNK), jnp.float32),
            pltpu.VMEM_SHARED((NSUB, NPAD), jnp.float32),
            pltpu.SemaphoreType.DMA,
        ],
    )
    def k(ei_hbm, out_hbm, dv, hist, red, tbuf, stag, semi):
        cid = lax.axis_index("c")
        sid = lax.axis_index("s")
        start, count = _my_chunks(sid)
        cbase = cid * CHUNKS_PER_CORE + start
        abase = (cbase // 8) * 8
        ofs = cbase - abase

        pltpu.async_copy(ei_hbm.at[1, pl.ds(abase, DEG_IDXROWS)], dv, semi)

        @pl.loop(0, NPAD // 16)
        def _(i):
            hist[pl.ds(i * 16, 16)] = jnp.zeros((16,), jnp.float32)

        pltpu.make_async_copy(ei_hbm.at[1, pl.ds(abase, DEG_IDXROWS)], dv, semi).wait()

        ones = jnp.full((16,), 1.0, jnp.float32)

        @pl.loop(0, count)
        def _(j):
            for sl in range(CHUNK // 16):
                idxv = dv[ofs + j, pl.ds(sl * 16, 16)]
                plsc.addupdate_scatter(hist, [idxv], ones)

        pltpu.sync_copy(hist, stag.at[sid])
        plsc.subcore_barrier()

        cols0 = jnp.zeros((16,), jnp.int32)
        lane = jnp.arange(16, dtype=jnp.int32)

        @pl.loop(0, DEG_BLK_PER_SUB)
        def _(b):
            blk = sid * DEG_BLK_PER_SUB + b
            pltpu.sync_copy(stag.at[:, pl.ds(blk * CHUNK, CHUNK)], red)
            for sl in range(CHUNK // 16):
                v = red[0, pl.ds(sl * 16, 16)]
                for r in range(1, NSUB):
                    v = v + red[r, pl.ds(sl * 16, 16)]
                plsc.store_scatter(tbuf, [sl * 16 + lane, cols0], v)
            pltpu.sync_copy(tbuf, out_hbm.at[cid, pl.ds(blk * CHUNK, CHUNK)])

    return k(ei2d)


@jax.jit
def _sc_aggregate(h, ei2d):
    @functools.partial(
        pl.kernel,
        out_type=jax.ShapeDtypeStruct((NCORE, NPAD, D), jnp.float32),
        mesh=_mesh(),
        scratch_types=[
            pltpu.VMEM((IDXROWS, CHUNK), jnp.int32),
            pltpu.VMEM((IDXROWS, CHUNK), jnp.int32),
            pltpu.VMEM((CHUNK, D), jnp.float32),
            pltpu.VMEM((CHUNK, D), jnp.float32),
            pltpu.VMEM_SHARED((NPAD, D), jnp.float32),
            pltpu.SemaphoreType.DMA,
            pltpu.SemaphoreType.DMA,
            pltpu.SemaphoreType.DMA,
        ],
    )
    def k(h_hbm, ei_hbm, out_hbm, sv, dv, buf0, buf1, acc,
          semi, sem0, sem1):
        cid = lax.axis_index("c")
        sid = lax.axis_index("s")

        start, count = _my_chunks(sid)
        cbase = cid * CHUNKS_PER_CORE + start
        abase = (cbase // 8) * 8
        ofs = cbase - abase

        def g_start(j, buf, sem):
            pltpu.async_copy(h_hbm.at[sv.at[ofs + j]], buf, sem)

        def g_wait(j, buf, sem):
            pltpu.make_async_copy(h_hbm.at[sv.at[ofs + j]], buf, sem).wait()

        def s_do(j, buf):
            pltpu.sync_copy(buf, acc.at[dv.at[ofs + j]], add=True)

        pltpu.async_copy(ei_hbm.at[0, pl.ds(abase, IDXROWS)], sv, semi)
        pltpu.async_copy(ei_hbm.at[1, pl.ds(abase, IDXROWS)], dv, semi)

        @pl.loop(0, ZROWS)
        def _(r):
            @pl.loop(0, D // 16)
            def _(cc):
                buf1[r, pl.ds(cc * 16, 16)] = jnp.zeros((16,), jnp.float32)

        pltpu.make_async_copy(ei_hbm.at[0, pl.ds(abase, IDXROWS)], sv, semi).wait()
        pltpu.make_async_copy(ei_hbm.at[1, pl.ds(abase, IDXROWS)], dv, semi).wait()
        g_start(0, buf0, sem0)

        @pl.loop(0, ROWS_PER_SUB // ZROWS)
        def _(j):
            pltpu.sync_copy(buf1, acc.at[pl.ds(sid * ROWS_PER_SUB + j * ZROWS, ZROWS)])

        plsc.subcore_barrier()

        for p in range(2):
            pbase = p * PHASE
            pcount = jnp.minimum(count - pbase, PHASE)
            if p > 0:
                pltpu.async_copy(ei_hbm.at[0, pl.ds(abase + pbase, IDXROWS)], sv, semi)
                pltpu.async_copy(ei_hbm.at[1, pl.ds(abase + pbase, IDXROWS)], dv, semi)
                pltpu.make_async_copy(ei_hbm.at[0, pl.ds(abase + pbase, IDXROWS)], sv, semi).wait()
                pltpu.make_async_copy(ei_hbm.at[1, pl.ds(abase + pbase, IDXROWS)], dv, semi).wait()

            npairs = pcount // 2
            tail = pcount - 2 * npairs

            if p > 0:
                g_start(0, buf0, sem0)

            @pl.loop(0, npairs)
            def _(t):
                c0 = 2 * t
                c1 = c0 + 1
                g_start(c1, buf1, sem1)
                g_wait(c0, buf0, sem0)
                s_do(c0, buf0)

                @pl.when(c1 + 1 < pcount)
                def _():
                    g_start(c1 + 1, buf0, sem0)

                g_wait(c1, buf1, sem1)
                s_do(c1, buf1)

            @pl.when(tail == 1)
            def _():
                c = 2 * npairs
                g_wait(c, buf0, sem0)
                s_do(c, buf0)

        plsc.subcore_barrier()

        @pl.loop(0, ROWS_PER_SUB // ZROWS)
        def _(j):
            r0 = sid * ROWS_PER_SUB + j * ZROWS
            pltpu.sync_copy(acc.at[pl.ds(r0, ZROWS)], out_hbm.at[cid, pl.ds(r0, ZROWS)])

    return k(h, ei2d)


def _tc_prep(x, W0):
    def body(x_ref, w_ref, h_ref):
        h_ref[...] = jnp.dot(x_ref[...], w_ref[...], preferred_element_type=jnp.float32)

    return pl.pallas_call(
        body,
        out_shape=jax.ShapeDtypeStruct((N, D), jnp.float32),
    )(x, W0)


def _tc_scale(hraw, degp):
    def body(h_ref, degp_ref, dinv_ref, hs_ref):
        deg = degp_ref[0, :N, 0:1] + degp_ref[1, :N, 0:1] + 1.0
        dinv = lax.rsqrt(deg)
        dinv_ref[...] = dinv
        hs_ref[...] = h_ref[...] * dinv

    return pl.pallas_call(
        body,
        out_shape=(
            jax.ShapeDtypeStruct((N, 1), jnp.float32),
            jax.ShapeDtypeStruct((N, D), jnp.float32),
        ),
    )(hraw, degp)


def _tc_combine(p, hp, dinv, b, g, be, resid, Wn, relu):
    have_res = resid is not None
    have_w = Wn is not None

    def body(*refs):
        it = iter(refs)
        p_ref = next(it)
        hp_ref = next(it)
        dinv_ref = next(it)
        b_ref = next(it)
        g_ref = next(it)
        be_ref = next(it)
        res_ref = next(it) if have_res else None
        w_ref = next(it) if have_w else None
        y_ref = next(it)
        hn_ref = next(it) if have_w else None

        dinv = dinv_ref[...]
        s = (p_ref[0, :N] + p_ref[1, :N] + hp_ref[...]) * dinv + b_ref[...]
        mu = jnp.mean(s, axis=0, keepdims=True)
        c = s - mu
        var = jnp.mean(c * c, axis=0, keepdims=True)
        y = g_ref[...] * c * lax.rsqrt(var + 1e-5) + be_ref[...]
        if relu:
            y = jnp.maximum(y, 0.0)
        if have_res:
            y = y + res_ref[...]
        y_ref[...] = y
        if have_w:
            hn = jnp.dot(y, w_ref[...], preferred_element_type=jnp.float32)
            hn_ref[...] = hn * dinv

    out_shape = [jax.ShapeDtypeStruct((N, D), jnp.float32)]
    if have_w:
        out_shape.append(jax.ShapeDtypeStruct((N, D), jnp.float32))
    args = [p, hp, dinv, b.reshape(1, D), g.reshape(1, D), be.reshape(1, D)]
    if have_res:
        args.append(resid)
    if have_w:
        args.append(Wn)
    res = pl.pallas_call(body, out_shape=tuple(out_shape))(*args)
    return res if have_w else (res[0], None)


def kernel(x, edge_index, W0, b0, g0, be0, W1, b1, g1, be1, W2, b2, g2, be2):
    ei = edge_index.astype(jnp.int32)
    pad = CH_PAD * CHUNK - E
    ei2d = jnp.pad(ei, ((0, 0), (0, pad))).reshape(2, CH_PAD, CHUNK)

    h0raw = _tc_prep(x, W0)
    degp = _sc_degree(ei2d)
    dinv, h0 = _tc_scale(h0raw, degp)

    p = _sc_aggregate(h0, ei2d)
    y0, h1 = _tc_combine(p, h0, dinv, b0, g0, be0, None, W1, relu=True)

    p = _sc_aggregate(h1, ei2d)
    y1, h2 = _tc_combine(p, h1, dinv, b1, g1, be1, y0, W2, relu=True)

    p = _sc_aggregate(h2, ei2d)
    y2, _ = _tc_combine(p, h2, dinv, b2, g2, be2, y1, None, relu=False)
    return y2
